# Initial kernel scaffold; baseline (speedup 1.0000x reference)
import jax
import jax.numpy as jnp
from jax.experimental import pallas as pl


def kernel(x, edge_index, W1, b1, W2, b2, W3, b3, fc_W, fc_b):
    raise NotImplementedError("write your pallas kernel here")


# speedup vs baseline: 21.4880x; 21.4880x over previous
import functools

import jax
import jax.numpy as jnp
from jax import lax
from jax.experimental import pallas as pl
from jax.experimental.pallas import tpu as pltpu
from jax.experimental.pallas import tpu_sc as plsc

N = 10000
E = 320000
D_IN = 128
D_HID = 64

NC = 2
NS = 16
NW = NC * NS
EW = E // NW
K = 80
G = EW // K
NP = 10240
RPS = NP // NS
DPS = RPS

_mesh = plsc.VectorSubcoreMesh(
    core_axis_name="c", subcore_axis_name="s", num_cores=NC, num_subcores=NS
)


def _wid():
    return lax.axis_index("s") * NC + lax.axis_index("c")


@functools.partial(
    pl.kernel,
    out_type=jax.ShapeDtypeStruct((NC * NP,), jnp.float32),
    mesh=_mesh,
    compiler_params=pltpu.CompilerParams(use_tc_tiling_on_sc=False),
    scratch_types=[
        pltpu.VMEM((G, K), jnp.int32),
        pltpu.VMEM((K,), jnp.float32),
        pltpu.VMEM_SHARED((NP,), jnp.float32),
    ],
)
def _deg_kernel(dst_hbm, zeros_hbm, out_hbm, dst_v, ones_v, deg_sh):
    cid = lax.axis_index("c")
    sid = lax.axis_index("s")
    wid = _wid()
    for i in range(K // 16):
        ones_v[pl.ds(i * 16, 16)] = jnp.ones((16,), jnp.float32)
    pltpu.sync_copy(zeros_hbm.at[pl.ds(sid * DPS, DPS)],
                    deg_sh.at[pl.ds(sid * DPS, DPS)])
    pltpu.sync_copy(dst_hbm.at[wid], dst_v)
    plsc.subcore_barrier()

    def body(g, carry):
        pltpu.sync_copy(ones_v, deg_sh.at[dst_v.at[g]], add=True)
        return carry

    lax.fori_loop(0, G, body, 0)
    plsc.subcore_barrier()
    pltpu.sync_copy(deg_sh.at[pl.ds(sid * DPS, DPS)],
                    out_hbm.at[pl.ds(cid * NP + sid * DPS, DPS)])


@functools.partial(
    pl.kernel,
    out_type=jax.ShapeDtypeStruct((NC, NP, D_HID), jnp.float32),
    mesh=_mesh,
    compiler_params=pltpu.CompilerParams(use_tc_tiling_on_sc=False),
    scratch_types=[
        pltpu.VMEM((G, K), jnp.int32),
        pltpu.VMEM((G, K), jnp.int32),
        pltpu.VMEM((K, D_HID), jnp.float32),
        pltpu.SemaphoreType.DMA,
        pltpu.VMEM_SHARED((NP, D_HID), jnp.float32),
    ],
)
def _agg_kernel(hs_hbm, src_hbm, dst_hbm, zeros_hbm, out_hbm,
                src_v, dst_v, rows_v, sem, acc_sh):
    cid = lax.axis_index("c")
    sid = lax.axis_index("s")
    wid = _wid()
    pltpu.sync_copy(zeros_hbm.at[pl.ds(sid * RPS, RPS)],
                    acc_sh.at[pl.ds(sid * RPS, RPS)])
    pltpu.sync_copy(src_hbm.at[wid], src_v)
    pltpu.sync_copy(dst_hbm.at[wid], dst_v)
    plsc.subcore_barrier()

    def body(g, carry):
        pltpu.async_copy(hs_hbm.at[src_v.at[g]], rows_v, sem).wait()
        pltpu.sync_copy(rows_v, acc_sh.at[dst_v.at[g]], add=True)
        return carry

    lax.fori_loop(0, G, body, 0)
    plsc.subcore_barrier()
    pltpu.sync_copy(acc_sh.at[pl.ds(sid * RPS, RPS)],
                    out_hbm.at[cid, pl.ds(sid * RPS, RPS)])


def _dis_body(degp_ref, out_ref):
    deg = degp_ref[:N] + degp_ref[NP:NP + N] + 1.0
    dis = lax.rsqrt(jnp.maximum(deg, 1e-12))
    out_ref[...] = jnp.broadcast_to(dis[:, None], (N, D_HID))


_dis_call = pl.pallas_call(
    _dis_body, out_shape=jax.ShapeDtypeStruct((N, D_HID), jnp.float32)
)


def _l1_body(x_ref, w_ref, dis_ref, out_ref):
    h = jnp.dot(x_ref[...], w_ref[...], preferred_element_type=jnp.float32)
    out_ref[...] = h * dis_ref[...]


_l1_call = pl.pallas_call(
    _l1_body, out_shape=jax.ShapeDtypeStruct((N, D_HID), jnp.float32)
)


def _mid_body(aggp_ref, hs_ref, dis_ref, b_ref, w_ref, out_ref):
    pre = dis_ref[...] * (aggp_ref[0, :N] + aggp_ref[1, :N] + hs_ref[...]) \
        + b_ref[...][None, :]
    act = jnp.maximum(pre, 0.0)
    h = jnp.dot(act, w_ref[...], preferred_element_type=jnp.float32)
    out_ref[...] = h * dis_ref[...]


_mid_call = pl.pallas_call(
    _mid_body, out_shape=jax.ShapeDtypeStruct((N, D_HID), jnp.float32)
)


def _fin_body(aggp_ref, hs_ref, dis_ref, b_ref, fcw_ref, fcb_ref, out_ref):
    h = dis_ref[...] * (aggp_ref[0, :N] + aggp_ref[1, :N] + hs_ref[...]) \
        + b_ref[...][None, :]
    g = jnp.sum(h, axis=0, keepdims=True) * (1.0 / N)
    out_ref[...] = jnp.dot(g, fcw_ref[...],
                           preferred_element_type=jnp.float32) + fcb_ref[...]


_fin_call = pl.pallas_call(
    _fin_body, out_shape=jax.ShapeDtypeStruct((1, 1), jnp.float32)
)


def kernel(x, edge_index, W1, b1, W2, b2, W3, b3, fc_W, fc_b):
    src = edge_index[0].reshape(NW, G, K)
    dst = edge_index[1].reshape(NW, G, K)
    zeros_rows = jnp.zeros((NP, D_HID), jnp.float32)
    zeros_deg = jnp.zeros((NC * NP,), jnp.float32)

    degp = _deg_kernel(dst, zeros_deg)
    dis = _dis_call(degp)

    hs1 = _l1_call(x, W1, dis)
    agg1 = _agg_kernel(hs1, src, dst, zeros_rows)
    hs2 = _mid_call(agg1, hs1, dis, b1, W2)
    agg2 = _agg_kernel(hs2, src, dst, zeros_rows)
    hs3 = _mid_call(agg2, hs2, dis, b2, W3)
    agg3 = _agg_kernel(hs3, src, dst, zeros_rows)
    return _fin_call(agg3, hs3, dis, b3, fc_W, fc_b.reshape(1, 1))


# speedup vs baseline: 38.9936x; 1.8147x over previous
import functools

import jax
import jax.numpy as jnp
from jax import lax
from jax.experimental import pallas as pl
from jax.experimental.pallas import tpu as pltpu
from jax.experimental.pallas import tpu_sc as plsc

N = 10000
E = 320000
D_IN = 128
D_HID = 64

NC = 2
NS = 16
NW = NC * NS
EW = E // NW
K = 125
G = EW // K
NB = 4
OPAD = 128
NP = 10240
RPS = NP // NS
DPS = RPS

_mesh = plsc.VectorSubcoreMesh(
    core_axis_name="c", subcore_axis_name="s", num_cores=NC, num_subcores=NS
)


def _wid():
    return lax.axis_index("s") * NC + lax.axis_index("c")


@functools.partial(
    pl.kernel,
    out_type=jax.ShapeDtypeStruct((NC * NP,), jnp.float32),
    mesh=_mesh,
    compiler_params=pltpu.CompilerParams(use_tc_tiling_on_sc=False),
    scratch_types=[
        pltpu.VMEM((G, K), jnp.int32),
        pltpu.VMEM((OPAD,), jnp.float32),
        pltpu.VMEM_SHARED((NP,), jnp.float32),
    ],
)
def _deg_kernel(dst_hbm, zeros_hbm, out_hbm, dst_v, ones_v, deg_sh):
    cid = lax.axis_index("c")
    sid = lax.axis_index("s")
    wid = _wid()
    for i in range(OPAD // 16):
        ones_v[pl.ds(i * 16, 16)] = jnp.ones((16,), jnp.float32)
    pltpu.sync_copy(zeros_hbm.at[pl.ds(sid * DPS, DPS)],
                    deg_sh.at[pl.ds(sid * DPS, DPS)])
    pltpu.sync_copy(dst_hbm.at[wid], dst_v)
    plsc.subcore_barrier()

    def body(g, carry):
        pltpu.sync_copy(ones_v.at[pl.ds(0, K)], deg_sh.at[dst_v.at[g]], add=True)
        return carry

    lax.fori_loop(0, G, body, 0)
    plsc.subcore_barrier()
    pltpu.sync_copy(deg_sh.at[pl.ds(sid * DPS, DPS)],
                    out_hbm.at[pl.ds(cid * NP + sid * DPS, DPS)])


@functools.partial(
    pl.kernel,
    out_type=jax.ShapeDtypeStruct((NC, NP, D_HID), jnp.float32),
    mesh=_mesh,
    compiler_params=pltpu.CompilerParams(use_tc_tiling_on_sc=False),
    scratch_types=[
        pltpu.VMEM((G, K), jnp.int32),
        pltpu.VMEM((G, K), jnp.int32),
        [pltpu.VMEM((K, D_HID), jnp.float32) for _ in range(NB)],
        [pltpu.SemaphoreType.DMA for _ in range(NB)],
        [pltpu.SemaphoreType.DMA for _ in range(NB)],
        pltpu.VMEM_SHARED((NP, D_HID), jnp.float32),
    ],
)
def _agg_kernel(hs_hbm, src_hbm, dst_hbm, zeros_hbm, out_hbm,
                src_v, dst_v, rows, gsem, ssem, acc_sh):
    cid = lax.axis_index("c")
    sid = lax.axis_index("s")
    wid = _wid()
    pltpu.sync_copy(zeros_hbm.at[pl.ds(sid * RPS, RPS)],
                    acc_sh.at[pl.ds(sid * RPS, RPS)])
    pltpu.sync_copy(src_hbm.at[wid], src_v)
    pltpu.sync_copy(dst_hbm.at[wid], dst_v)
    plsc.subcore_barrier()

    def fire_gather(b, g):
        pltpu.async_copy(hs_hbm.at[src_v.at[g]], rows[b], gsem[b])

    def wait_gather(b):
        pltpu.make_async_copy(hs_hbm.at[src_v.at[0]], rows[b], gsem[b]).wait()

    def fire_scatter(b, g):
        pltpu.async_copy(rows[b], acc_sh.at[dst_v.at[g]], ssem[b], add=True)

    def wait_scatter(b):
        pltpu.make_async_copy(rows[b], acc_sh.at[dst_v.at[0]], ssem[b]).wait()

    for b in range(NB):
        fire_gather(b, b)

    def outer(o, carry):
        for b in range(NB):
            wait_gather(b)
            fire_scatter(b, o * NB + b)
        for b in range(NB):
            wait_scatter(b)
            fire_gather(b, (o + 1) * NB + b)
        return carry

    lax.fori_loop(0, G // NB - 1, outer, 0)
    for b in range(NB):
        wait_gather(b)
        fire_scatter(b, G - NB + b)
    for b in range(NB):
        wait_scatter(b)
    plsc.subcore_barrier()
    pltpu.sync_copy(acc_sh.at[pl.ds(sid * RPS, RPS)],
                    out_hbm.at[cid, pl.ds(sid * RPS, RPS)])


def _dis_body(degp_ref, out_ref):
    deg = degp_ref[:N] + degp_ref[NP:NP + N] + 1.0
    dis = lax.rsqrt(jnp.maximum(deg, 1e-12))
    out_ref[...] = jnp.broadcast_to(dis[:, None], (N, D_HID))


_dis_call = pl.pallas_call(
    _dis_body, out_shape=jax.ShapeDtypeStruct((N, D_HID), jnp.float32)
)


def _l1_body(x_ref, w_ref, dis_ref, out_ref):
    h = jnp.dot(x_ref[...], w_ref[...], preferred_element_type=jnp.float32)
    out_ref[...] = h * dis_ref[...]


_l1_call = pl.pallas_call(
    _l1_body, out_shape=jax.ShapeDtypeStruct((N, D_HID), jnp.float32)
)


def _mid_body(aggp_ref, hs_ref, dis_ref, b_ref, w_ref, out_ref):
    pre = dis_ref[...] * (aggp_ref[0, :N] + aggp_ref[1, :N] + hs_ref[...]) \
        + b_ref[...][None, :]
    act = jnp.maximum(pre, 0.0)
    h = jnp.dot(act, w_ref[...], preferred_element_type=jnp.float32)
    out_ref[...] = h * dis_ref[...]


_mid_call = pl.pallas_call(
    _mid_body, out_shape=jax.ShapeDtypeStruct((N, D_HID), jnp.float32)
)


def _fin_body(aggp_ref, hs_ref, dis_ref, b_ref, fcw_ref, fcb_ref, out_ref):
    h = dis_ref[...] * (aggp_ref[0, :N] + aggp_ref[1, :N] + hs_ref[...]) \
        + b_ref[...][None, :]
    g = jnp.sum(h, axis=0, keepdims=True) * (1.0 / N)
    out_ref[...] = jnp.dot(g, fcw_ref[...],
                           preferred_element_type=jnp.float32) + fcb_ref[...]


_fin_call = pl.pallas_call(
    _fin_body, out_shape=jax.ShapeDtypeStruct((1, 1), jnp.float32)
)


def kernel(x, edge_index, W1, b1, W2, b2, W3, b3, fc_W, fc_b):
    src = edge_index[0].reshape(NW, G, K)
    dst = edge_index[1].reshape(NW, G, K)
    zeros_rows = jnp.zeros((NP, D_HID), jnp.float32)
    zeros_deg = jnp.zeros((NC * NP,), jnp.float32)

    degp = _deg_kernel(dst, zeros_deg)
    dis = _dis_call(degp)

    hs1 = _l1_call(x, W1, dis)
    agg1 = _agg_kernel(hs1, src, dst, zeros_rows)
    hs2 = _mid_call(agg1, hs1, dis, b1, W2)
    agg2 = _agg_kernel(hs2, src, dst, zeros_rows)
    hs3 = _mid_call(agg2, hs2, dis, b2, W3)
    agg3 = _agg_kernel(hs3, src, dst, zeros_rows)
    return _fin_call(agg3, hs3, dis, b3, fc_W, fc_b.reshape(1, 1))


# speedup vs baseline: 40.9818x; 1.0510x over previous
import functools

import jax
import jax.numpy as jnp
from jax import lax
from jax.experimental import pallas as pl
from jax.experimental.pallas import tpu as pltpu
from jax.experimental.pallas import tpu_sc as plsc

N = 10000
E = 320000
D_IN = 128
D_HID = 64

NC = 2
NS = 16
NW = NC * NS
EW = E // NW
K = 125
G = EW // K
NB = 8
OPAD = 128
NP = 10240
RPS = NP // NS
DPS = RPS

_mesh = plsc.VectorSubcoreMesh(
    core_axis_name="c", subcore_axis_name="s", num_cores=NC, num_subcores=NS
)


def _wid():
    return lax.axis_index("s") * NC + lax.axis_index("c")


@functools.partial(
    pl.kernel,
    out_type=jax.ShapeDtypeStruct((NC * NP,), jnp.float32),
    mesh=_mesh,
    compiler_params=pltpu.CompilerParams(use_tc_tiling_on_sc=False),
    scratch_types=[
        pltpu.VMEM((G, K), jnp.int32),
        pltpu.VMEM((OPAD,), jnp.float32),
        pltpu.VMEM_SHARED((NP,), jnp.float32),
    ],
)
def _deg_kernel(dst_hbm, zeros_hbm, out_hbm, dst_v, ones_v, deg_sh):
    cid = lax.axis_index("c")
    sid = lax.axis_index("s")
    wid = _wid()
    for i in range(OPAD // 16):
        ones_v[pl.ds(i * 16, 16)] = jnp.ones((16,), jnp.float32)
    pltpu.sync_copy(zeros_hbm.at[pl.ds(sid * DPS, DPS)],
                    deg_sh.at[pl.ds(sid * DPS, DPS)])
    pltpu.sync_copy(dst_hbm.at[wid], dst_v)
    plsc.subcore_barrier()

    def body(g, carry):
        pltpu.sync_copy(ones_v.at[pl.ds(0, K)], deg_sh.at[dst_v.at[g]], add=True)
        return carry

    lax.fori_loop(0, G, body, 0)
    plsc.subcore_barrier()
    pltpu.sync_copy(deg_sh.at[pl.ds(sid * DPS, DPS)],
                    out_hbm.at[pl.ds(cid * NP + sid * DPS, DPS)])


@functools.partial(
    pl.kernel,
    out_type=jax.ShapeDtypeStruct((NC, NP, D_HID), jnp.float32),
    mesh=_mesh,
    compiler_params=pltpu.CompilerParams(use_tc_tiling_on_sc=False),
    scratch_types=[
        pltpu.VMEM((G, K), jnp.int32),
        pltpu.VMEM((G, K), jnp.int32),
        [pltpu.VMEM((K, D_HID), jnp.float32) for _ in range(NB)],
        [pltpu.SemaphoreType.DMA for _ in range(NB)],
        [pltpu.SemaphoreType.DMA for _ in range(NB)],
        pltpu.VMEM_SHARED((NP, D_HID), jnp.float32),
    ],
)
def _agg_kernel(hs_hbm, src_hbm, dst_hbm, zeros_hbm, out_hbm,
                src_v, dst_v, rows, gsem, ssem, acc_sh):
    cid = lax.axis_index("c")
    sid = lax.axis_index("s")
    wid = _wid()
    pltpu.sync_copy(zeros_hbm.at[pl.ds(sid * RPS, RPS)],
                    acc_sh.at[pl.ds(sid * RPS, RPS)])
    pltpu.sync_copy(src_hbm.at[wid], src_v)
    pltpu.sync_copy(dst_hbm.at[wid], dst_v)
    plsc.subcore_barrier()

    def fire_gather(b, g):
        pltpu.async_copy(hs_hbm.at[src_v.at[g]], rows[b], gsem[b])

    def wait_gather(b):
        pltpu.make_async_copy(hs_hbm.at[src_v.at[0]], rows[b], gsem[b]).wait()

    def fire_scatter(b, g):
        pltpu.async_copy(rows[b], acc_sh.at[dst_v.at[g]], ssem[b], add=True)

    def wait_scatter(b):
        pltpu.make_async_copy(rows[b], acc_sh.at[dst_v.at[0]], ssem[b]).wait()

    for b in range(NB):
        fire_gather(b, b)

    def outer(o, carry):
        for b in range(NB):
            wait_gather(b)
            fire_scatter(b, o * NB + b)
        for b in range(NB):
            wait_scatter(b)
            fire_gather(b, (o + 1) * NB + b)
        return carry

    lax.fori_loop(0, G // NB - 1, outer, 0)
    for b in range(NB):
        wait_gather(b)
        fire_scatter(b, G - NB + b)
    for b in range(NB):
        wait_scatter(b)
    plsc.subcore_barrier()
    pltpu.sync_copy(acc_sh.at[pl.ds(sid * RPS, RPS)],
                    out_hbm.at[cid, pl.ds(sid * RPS, RPS)])


def _l1_body(x_ref, w_ref, degp_ref, hs_ref, dis_ref):
    deg = degp_ref[:N] + degp_ref[NP:NP + N] + 1.0
    dis = lax.rsqrt(jnp.maximum(deg, 1e-12))
    dis_b = jnp.broadcast_to(dis[:, None], (N, D_HID))
    dis_ref[...] = dis_b
    h = jnp.dot(x_ref[...], w_ref[...], preferred_element_type=jnp.float32)
    hs_ref[...] = h * dis_b


_l1_call = pl.pallas_call(
    _l1_body,
    out_shape=[jax.ShapeDtypeStruct((N, D_HID), jnp.float32),
               jax.ShapeDtypeStruct((N, D_HID), jnp.float32)],
)


def _mid_body(aggp_ref, hs_ref, dis_ref, b_ref, w_ref, out_ref):
    pre = dis_ref[...] * (aggp_ref[0, :N] + aggp_ref[1, :N] + hs_ref[...]) \
        + b_ref[...][None, :]
    act = jnp.maximum(pre, 0.0)
    h = jnp.dot(act, w_ref[...], preferred_element_type=jnp.float32)
    out_ref[...] = h * dis_ref[...]


_mid_call = pl.pallas_call(
    _mid_body, out_shape=jax.ShapeDtypeStruct((N, D_HID), jnp.float32)
)


def _fin_body(aggp_ref, hs_ref, dis_ref, b_ref, fcw_ref, fcb_ref, out_ref):
    h = dis_ref[...] * (aggp_ref[0, :N] + aggp_ref[1, :N] + hs_ref[...]) \
        + b_ref[...][None, :]
    g = jnp.sum(h, axis=0, keepdims=True) * (1.0 / N)
    out_ref[...] = jnp.dot(g, fcw_ref[...],
                           preferred_element_type=jnp.float32) + fcb_ref[...]


_fin_call = pl.pallas_call(
    _fin_body, out_shape=jax.ShapeDtypeStruct((1, 1), jnp.float32)
)


def kernel(x, edge_index, W1, b1, W2, b2, W3, b3, fc_W, fc_b):
    src = edge_index[0].reshape(NW, G, K)
    dst = edge_index[1].reshape(NW, G, K)
    zeros_rows = jnp.zeros((NP, D_HID), jnp.float32)
    zeros_deg = jnp.zeros((NC * NP,), jnp.float32)

    degp = _deg_kernel(dst, zeros_deg)
    hs1, dis = _l1_call(x, W1, degp)
    agg1 = _agg_kernel(hs1, src, dst, zeros_rows)
    hs2 = _mid_call(agg1, hs1, dis, b1, W2)
    agg2 = _agg_kernel(hs2, src, dst, zeros_rows)
    hs3 = _mid_call(agg2, hs2, dis, b2, W3)
    agg3 = _agg_kernel(hs3, src, dst, zeros_rows)
    return _fin_call(agg3, hs3, dis, b3, fc_W, fc_b.reshape(1, 1))
